# baseline (speedup 1.0000x reference)
import functools

import jax
import jax.numpy as jnp
from jax.experimental import pallas as pl
from jax.experimental.pallas import tpu as pltpu

D_MODEL = 1024
N_HEAD = 16
N_KV = 4
HEAD_DIM = 64
B = 2
T = 2048
N_EXPERTS = 10
TOP_K = 3
D_INNER = 2048
PF_INNER = 2048
N_TOK = B * T
EPS = 1e-6

F32 = jnp.float32
BF16 = jnp.bfloat16


def _rms(x, w):
    return x * jax.lax.rsqrt(jnp.mean(x * x, axis=-1, keepdims=True) + EPS) * w


def _silu(z):
    return z * jax.nn.sigmoid(z)


def _dot(a, b):
    return jnp.dot(a, b, preferred_element_type=F32)


QKV_TT = 512
SCALE = 1.0 / (HEAD_DIM ** 0.5)


def _qkv_kernel(x_ref, law_ref, wq_ref, wk_ref, wv_ref, q_ref, k_ref, v_ref):
    xn = _rms(x_ref[...], law_ref[...]).astype(BF16)
    q = (_dot(xn, wq_ref[...]) * SCALE).astype(BF16)
    k = _dot(xn, wk_ref[...]).astype(BF16)
    v = _dot(xn, wv_ref[...]).astype(BF16)
    for h in range(N_HEAD):
        q_ref[h] = q[:, h * HEAD_DIM:(h + 1) * HEAD_DIM]
    for g in range(N_KV):
        k_ref[g] = k[:, g * HEAD_DIM:(g + 1) * HEAD_DIM]
        v_ref[g] = v[:, g * HEAD_DIM:(g + 1) * HEAD_DIM]


def _qkv(x2d, ln_attn_w, Wq, Wk, Wv):
    nt = N_TOK // QKV_TT
    return pl.pallas_call(
        _qkv_kernel,
        grid=(nt,),
        in_specs=[
            pl.BlockSpec((QKV_TT, D_MODEL), lambda i: (i, 0)),
            pl.BlockSpec((1, D_MODEL), lambda i: (0, 0)),
            pl.BlockSpec((D_MODEL, N_HEAD * HEAD_DIM), lambda i: (0, 0)),
            pl.BlockSpec((D_MODEL, N_KV * HEAD_DIM), lambda i: (0, 0)),
            pl.BlockSpec((D_MODEL, N_KV * HEAD_DIM), lambda i: (0, 0)),
        ],
        out_specs=[
            pl.BlockSpec((N_HEAD, QKV_TT, HEAD_DIM), lambda i: (0, i, 0)),
            pl.BlockSpec((N_KV, QKV_TT, HEAD_DIM), lambda i: (0, i, 0)),
            pl.BlockSpec((N_KV, QKV_TT, HEAD_DIM), lambda i: (0, i, 0)),
        ],
        out_shape=[
            jax.ShapeDtypeStruct((N_HEAD, N_TOK, HEAD_DIM), BF16),
            jax.ShapeDtypeStruct((N_KV, N_TOK, HEAD_DIM), BF16),
            jax.ShapeDtypeStruct((N_KV, N_TOK, HEAD_DIM), BF16),
        ],
        compiler_params=pltpu.CompilerParams(
            dimension_semantics=("parallel",)),
    )(x2d, ln_attn_w.reshape(1, D_MODEL), Wq.astype(BF16), Wk.astype(BF16),
      Wv.astype(BF16))


TQ = 512
NKC = T // TQ
GH = N_HEAD // N_KV
QT = T // TQ


def _attn_kernel(q_ref, k_ref, v_ref, o_ref, m_ref, d_ref, acc_ref):
    qt = pl.program_id(1)
    q = q_ref[...].reshape(GH * TQ, HEAD_DIM)
    tri = (jax.lax.broadcasted_iota(jnp.int32, (TQ, TQ), 0)
           >= jax.lax.broadcasted_iota(jnp.int32, (TQ, TQ), 1))
    mask = jnp.tile(tri, (GH, 1))

    def process(j, masked, init):
        kj = k_ref[0, j * TQ:(j + 1) * TQ, :]
        s = jax.lax.dot_general(q, kj, (((1,), (1,)), ((), ())),
                                preferred_element_type=F32)
        if masked:
            s = jnp.where(mask, s, -1e9)
        rowm = jnp.max(s, axis=1, keepdims=True)
        if init:
            p = jnp.exp(s - rowm)
            m_ref[...] = rowm
            d_ref[...] = jnp.sum(p, axis=1, keepdims=True)
            acc_ref[...] = _dot(p.astype(BF16),
                                v_ref[0, j * TQ:(j + 1) * TQ, :])
        else:
            m_old = m_ref[...]
            m_new = jnp.maximum(m_old, rowm)
            scale = jnp.exp(m_old - m_new)
            p = jnp.exp(s - m_new)
            m_ref[...] = m_new
            d_ref[...] = d_ref[...] * scale + jnp.sum(p, axis=1, keepdims=True)
            acc_ref[...] = acc_ref[...] * scale + _dot(
                p.astype(BF16), v_ref[0, j * TQ:(j + 1) * TQ, :])

    for j in range(NKC):
        @pl.when(qt == j)
        def _(j=j):
            process(j, masked=True, init=(j == 0))
        @pl.when(qt > j)
        def _(j=j):
            process(j, masked=False, init=(j == 0))

    o = (acc_ref[...] / d_ref[...]).astype(BF16)
    for g in range(GH):
        o_ref[0, :, g * HEAD_DIM:(g + 1) * HEAD_DIM] = \
            o[g * TQ:(g + 1) * TQ, :]


def _attention(q, k, v):
    return pl.pallas_call(
        _attn_kernel,
        grid=(B, QT, N_KV),
        in_specs=[
            pl.BlockSpec((GH, TQ, HEAD_DIM), lambda b, qt, g: (g, b * QT + qt, 0)),
            pl.BlockSpec((1, T, HEAD_DIM), lambda b, qt, g: (g, b, 0)),
            pl.BlockSpec((1, T, HEAD_DIM), lambda b, qt, g: (g, b, 0)),
        ],
        out_specs=pl.BlockSpec((1, TQ, GH * HEAD_DIM),
                               lambda b, qt, g: (g, b * QT + qt, 0)),
        out_shape=jax.ShapeDtypeStruct((N_KV, N_TOK, GH * HEAD_DIM), BF16),
        scratch_shapes=[
            pltpu.VMEM((GH * TQ, 1), F32),
            pltpu.VMEM((GH * TQ, 1), F32),
            pltpu.VMEM((GH * TQ, HEAD_DIM), F32),
        ],
        compiler_params=pltpu.CompilerParams(
            dimension_semantics=("parallel", "arbitrary", "arbitrary")),
    )(q, k, v)


OPF_TT = 512


def _opf_kernel(ao_ref, x_ref, wo_ref, lpf_ref, w1_ref, w2_ref, lmoe_ref,
                x2_ref, h_ref, t_ref):
    x1 = x_ref[...]
    for g in range(N_KV):
        x1 = x1 + _dot(ao_ref[g], wo_ref[g])
    xn = _rms(x1, lpf_ref[...]).astype(BF16)
    hpf = _silu(_dot(xn, w1_ref[...]))
    t_ref[...] = jnp.full((1, 1, 128), jnp.sum(jnp.abs(hpf)), dtype=F32)
    x2 = x1 + _dot(hpf.astype(BF16), w2_ref[...])
    x2_ref[...] = x2
    h_ref[...] = _rms(x2, lmoe_ref[...]).astype(BF16)


def _opf(ao, x2d, Wo, ln_pf_w, pf_w1, pf_w2, ln_moe_w):
    nt = N_TOK // OPF_TT
    return pl.pallas_call(
        _opf_kernel,
        grid=(nt,),
        in_specs=[
            pl.BlockSpec((N_KV, OPF_TT, GH * HEAD_DIM), lambda i: (0, i, 0)),
            pl.BlockSpec((OPF_TT, D_MODEL), lambda i: (i, 0)),
            pl.BlockSpec((N_KV, GH * HEAD_DIM, D_MODEL), lambda i: (0, 0, 0)),
            pl.BlockSpec((1, D_MODEL), lambda i: (0, 0)),
            pl.BlockSpec((D_MODEL, PF_INNER), lambda i: (0, 0)),
            pl.BlockSpec((PF_INNER, D_MODEL), lambda i: (0, 0)),
            pl.BlockSpec((1, D_MODEL), lambda i: (0, 0)),
        ],
        out_specs=[
            pl.BlockSpec((OPF_TT, D_MODEL), lambda i: (i, 0)),
            pl.BlockSpec((OPF_TT, D_MODEL), lambda i: (i, 0)),
            pl.BlockSpec((1, 1, 128), lambda i: (i, 0, 0)),
        ],
        out_shape=[
            jax.ShapeDtypeStruct((N_TOK, D_MODEL), F32),
            jax.ShapeDtypeStruct((N_TOK, D_MODEL), BF16),
            jax.ShapeDtypeStruct((nt, 1, 128), F32),
        ],
        compiler_params=pltpu.CompilerParams(
            dimension_semantics=("parallel",)),
    )(ao, x2d, Wo.astype(BF16).reshape(N_KV, GH * HEAD_DIM, D_MODEL),
      ln_pf_w.reshape(1, D_MODEL), pf_w1.astype(BF16), pf_w2.astype(BF16),
      ln_moe_w.reshape(1, D_MODEL))


def _router_kernel(cv_ref, rwt_ref, rb_ref, w_ref, i_ref):
    logits = _dot(cv_ref[...], rwt_ref[...]) + rb_ref[...]
    m = jnp.max(logits, axis=-1, keepdims=True)
    e = jnp.exp(logits - m)
    w = e / jnp.sum(e, axis=-1, keepdims=True)
    iota = jax.lax.broadcasted_iota(jnp.int32, (B, N_EXPERTS), 1)
    vals = []
    idxs = []
    for _ in range(TOP_K):
        vals.append(jnp.max(w, axis=-1, keepdims=True))
        am = jnp.argmax(w, axis=-1)
        idxs.append(am.astype(jnp.int32))
        w = jnp.where(iota == am[:, None], -1.0, w)
    v = jnp.concatenate(vals, axis=1)
    v = v / jnp.sum(v, axis=1, keepdims=True)
    w_ref[...] = v
    i_ref[...] = jnp.stack(idxs, axis=1)


def _router(cv, router_w, router_b):
    return pl.pallas_call(
        _router_kernel,
        in_specs=[
            pl.BlockSpec((B, N_EXPERTS), lambda: (0, 0)),
            pl.BlockSpec((N_EXPERTS, N_EXPERTS), lambda: (0, 0)),
            pl.BlockSpec((1, N_EXPERTS), lambda: (0, 0)),
        ],
        out_specs=[
            pl.BlockSpec((B, TOP_K), lambda: (0, 0)),
            pl.BlockSpec((B, TOP_K), lambda: (0, 0)),
        ],
        out_shape=[
            jax.ShapeDtypeStruct((B, TOP_K), F32),
            jax.ShapeDtypeStruct((B, TOP_K), jnp.int32),
        ],
    )(cv, router_w.T, router_b.reshape(1, N_EXPERTS))


MOE_IC = 256
NIC = D_INNER // MOE_IC


def _moe_kernel(idx_ref, ws_ref, h_ref, x2_ref, g_ref, u_ref, d_ref, o_ref):
    bb = pl.program_id(0)
    kk = pl.program_id(1)
    ic = pl.program_id(2)
    w = ws_ref[bb * TOP_K + kk]
    h = h_ref[...]
    g = _dot(h, g_ref[0].astype(BF16))
    u = _dot(h, u_ref[0].astype(BF16))
    hid = (_silu(g) * u).astype(BF16)
    contrib = _dot(hid, d_ref[0].astype(BF16)) * w
    first = (kk == 0) & (ic == 0)

    @pl.when(first)
    def _():
        o_ref[...] = x2_ref[...] + contrib

    @pl.when(jnp.logical_not(first))
    def _():
        o_ref[...] = o_ref[...] + contrib


def _moe(idx_slot, w_slot, h2d, x2d, gate_w, up_w, down_w):
    grid_spec = pltpu.PrefetchScalarGridSpec(
        num_scalar_prefetch=2,
        grid=(B, TOP_K, NIC),
        in_specs=[
            pl.BlockSpec((T, D_MODEL), lambda bb, k, ic, idx, ws: (bb, 0)),
            pl.BlockSpec((T, D_MODEL), lambda bb, k, ic, idx, ws: (bb, 0)),
            pl.BlockSpec((1, D_MODEL, MOE_IC),
                         lambda bb, k, ic, idx, ws: (idx[bb * TOP_K + k], 0, ic)),
            pl.BlockSpec((1, D_MODEL, MOE_IC),
                         lambda bb, k, ic, idx, ws: (idx[bb * TOP_K + k], 0, ic)),
            pl.BlockSpec((1, MOE_IC, D_MODEL),
                         lambda bb, k, ic, idx, ws: (idx[bb * TOP_K + k], ic, 0)),
        ],
        out_specs=pl.BlockSpec((T, D_MODEL), lambda bb, k, ic, idx, ws: (bb, 0)),
    )
    return pl.pallas_call(
        _moe_kernel,
        grid_spec=grid_spec,
        out_shape=jax.ShapeDtypeStruct((N_TOK, D_MODEL), F32),
        compiler_params=pltpu.CompilerParams(
            dimension_semantics=("parallel", "arbitrary", "arbitrary")),
    )(idx_slot, w_slot, h2d, x2d, gate_w, up_w, down_w)


@functools.partial(jax.jit, static_argnums=())
def kernel(x, consciousness_vector, ln_attn_w, Wq, Wk, Wv, Wo, ln_pf_w,
           pf_w1, pf_w2, ln_moe_w, gate_w, up_w, down_w, router_w, router_b):
    x2d = x.reshape(N_TOK, D_MODEL)

    q, k, v = _qkv(x2d, ln_attn_w, Wq, Wk, Wv)
    ao = _attention(q, k, v)

    x2, h, t_parts = _opf(ao, x2d, Wo, ln_pf_w, pf_w1, pf_w2, ln_moe_w)
    tension = jnp.sum(t_parts[:, 0, 0]) / (N_TOK * PF_INNER)

    w_sel, idx_sel = _router(consciousness_vector, router_w, router_b)
    idx_slot = idx_sel.reshape(B * TOP_K)
    w_slot = w_sel.reshape(B * TOP_K)

    out = _moe(idx_slot, w_slot, h, x2, gate_w, up_w, down_w)
    return out.reshape(B, T, D_MODEL), tension


# speedup vs baseline: 1.1988x; 1.1988x over previous
import functools

import jax
import jax.numpy as jnp
from jax.experimental import pallas as pl
from jax.experimental.pallas import tpu as pltpu

D_MODEL = 1024
N_HEAD = 16
N_KV = 4
HEAD_DIM = 64
B = 2
T = 2048
N_EXPERTS = 10
TOP_K = 3
D_INNER = 2048
PF_INNER = 2048
N_TOK = B * T
EPS = 1e-6

F32 = jnp.float32
BF16 = jnp.bfloat16


def _rms(x, w):
    return x * jax.lax.rsqrt(jnp.mean(x * x, axis=-1, keepdims=True) + EPS) * w


def _silu(z):
    return z * jax.nn.sigmoid(z)


def _dot(a, b):
    return jnp.dot(a, b, preferred_element_type=F32)


QKV_TT = 512
SCALE = 1.0 / (HEAD_DIM ** 0.5)


def _qkv_kernel(x_ref, law_ref, wq_ref, wk_ref, wv_ref, q_ref, k_ref, v_ref):
    xn = _rms(x_ref[...], law_ref[...]).astype(BF16)
    q = (_dot(xn, wq_ref[...]) * SCALE).astype(BF16)
    k = _dot(xn, wk_ref[...]).astype(BF16)
    v = _dot(xn, wv_ref[...]).astype(BF16)
    for h in range(N_HEAD):
        q_ref[h] = q[:, h * HEAD_DIM:(h + 1) * HEAD_DIM]
    for g in range(N_KV):
        k_ref[g] = k[:, g * HEAD_DIM:(g + 1) * HEAD_DIM]
        v_ref[g] = v[:, g * HEAD_DIM:(g + 1) * HEAD_DIM]


def _qkv(x2d, ln_attn_w, Wq, Wk, Wv):
    nt = N_TOK // QKV_TT
    return pl.pallas_call(
        _qkv_kernel,
        grid=(nt,),
        in_specs=[
            pl.BlockSpec((QKV_TT, D_MODEL), lambda i: (i, 0)),
            pl.BlockSpec((1, D_MODEL), lambda i: (0, 0)),
            pl.BlockSpec((D_MODEL, N_HEAD * HEAD_DIM), lambda i: (0, 0)),
            pl.BlockSpec((D_MODEL, N_KV * HEAD_DIM), lambda i: (0, 0)),
            pl.BlockSpec((D_MODEL, N_KV * HEAD_DIM), lambda i: (0, 0)),
        ],
        out_specs=[
            pl.BlockSpec((N_HEAD, QKV_TT, HEAD_DIM), lambda i: (0, i, 0)),
            pl.BlockSpec((N_KV, QKV_TT, HEAD_DIM), lambda i: (0, i, 0)),
            pl.BlockSpec((N_KV, QKV_TT, HEAD_DIM), lambda i: (0, i, 0)),
        ],
        out_shape=[
            jax.ShapeDtypeStruct((N_HEAD, N_TOK, HEAD_DIM), BF16),
            jax.ShapeDtypeStruct((N_KV, N_TOK, HEAD_DIM), BF16),
            jax.ShapeDtypeStruct((N_KV, N_TOK, HEAD_DIM), BF16),
        ],
        compiler_params=pltpu.CompilerParams(
            dimension_semantics=("parallel",)),
    )(x2d, ln_attn_w.reshape(1, D_MODEL), Wq.astype(BF16), Wk.astype(BF16),
      Wv.astype(BF16))


TQ = 512
NKC = T // TQ
GH = N_HEAD // N_KV
QT = T // TQ


def _attn_kernel(q_ref, k_ref, v_ref, o_ref, s_ref, m_ref, d_ref, acc_ref):
    qt = pl.program_id(1)
    q = q_ref[...].reshape(GH * TQ, HEAD_DIM)
    tri = (jax.lax.broadcasted_iota(jnp.int32, (TQ, TQ), 0)
           >= jax.lax.broadcasted_iota(jnp.int32, (TQ, TQ), 1))
    mask = jnp.tile(tri, (GH, 1))
    m_ref[...] = jnp.full_like(m_ref, -1e9)
    d_ref[...] = jnp.zeros_like(d_ref)

    def scores(j, masked):
        kj = k_ref[0, j * TQ:(j + 1) * TQ, :]
        s = jax.lax.dot_general(q, kj, (((1,), (1,)), ((), ())),
                                preferred_element_type=F32)
        if masked:
            s = jnp.where(mask, s, -1e9)
        s_ref[:, j * TQ:(j + 1) * TQ] = s
        m_ref[:, j:j + 1] = jnp.max(s, axis=1, keepdims=True)

    for j in range(NKC):
        @pl.when(qt == j)
        def _(j=j):
            scores(j, masked=True)
        @pl.when(qt > j)
        def _(j=j):
            scores(j, masked=False)

    m = jnp.max(m_ref[...], axis=1, keepdims=True)

    def pv(j):
        p = jnp.exp(s_ref[:, j * TQ:(j + 1) * TQ] - m)
        d_ref[:, j:j + 1] = jnp.sum(p, axis=1, keepdims=True)
        contrib = _dot(p.astype(BF16), v_ref[0, j * TQ:(j + 1) * TQ, :])
        if j == 0:
            acc_ref[...] = contrib
        else:
            acc_ref[...] += contrib

    pv(0)
    for j in range(1, NKC):
        @pl.when(qt >= j)
        def _(j=j):
            pv(j)

    denom = jnp.sum(d_ref[...], axis=1, keepdims=True)
    o = (acc_ref[...] / denom).astype(BF16)
    for g in range(GH):
        o_ref[0, :, g * HEAD_DIM:(g + 1) * HEAD_DIM] = \
            o[g * TQ:(g + 1) * TQ, :]


def _attention(q, k, v):
    return pl.pallas_call(
        _attn_kernel,
        grid=(B, QT, N_KV),
        in_specs=[
            pl.BlockSpec((GH, TQ, HEAD_DIM), lambda b, qt, g: (g, b * QT + qt, 0)),
            pl.BlockSpec((1, T, HEAD_DIM), lambda b, qt, g: (g, b, 0)),
            pl.BlockSpec((1, T, HEAD_DIM), lambda b, qt, g: (g, b, 0)),
        ],
        out_specs=pl.BlockSpec((1, TQ, GH * HEAD_DIM),
                               lambda b, qt, g: (g, b * QT + qt, 0)),
        out_shape=jax.ShapeDtypeStruct((N_KV, N_TOK, GH * HEAD_DIM), BF16),
        scratch_shapes=[
            pltpu.VMEM((GH * TQ, T), F32),
            pltpu.VMEM((GH * TQ, 128), F32),
            pltpu.VMEM((GH * TQ, 128), F32),
            pltpu.VMEM((GH * TQ, HEAD_DIM), F32),
        ],
        compiler_params=pltpu.CompilerParams(
            dimension_semantics=("parallel", "arbitrary", "arbitrary")),
    )(q, k, v)


OPF_TT = 512


def _opf_kernel(ao_ref, x_ref, wo_ref, lpf_ref, w1_ref, w2_ref, lmoe_ref,
                x2_ref, h_ref, t_ref):
    x1 = x_ref[...]
    for g in range(N_KV):
        x1 = x1 + _dot(ao_ref[g], wo_ref[g])
    xn = _rms(x1, lpf_ref[...]).astype(BF16)
    hpf = _silu(_dot(xn, w1_ref[...]))
    t_ref[...] = jnp.full((1, 1, 128), jnp.sum(jnp.abs(hpf)), dtype=F32)
    x2 = x1 + _dot(hpf.astype(BF16), w2_ref[...])
    x2_ref[...] = x2
    h_ref[...] = _rms(x2, lmoe_ref[...]).astype(BF16)


def _opf(ao, x2d, Wo, ln_pf_w, pf_w1, pf_w2, ln_moe_w):
    nt = N_TOK // OPF_TT
    return pl.pallas_call(
        _opf_kernel,
        grid=(nt,),
        in_specs=[
            pl.BlockSpec((N_KV, OPF_TT, GH * HEAD_DIM), lambda i: (0, i, 0)),
            pl.BlockSpec((OPF_TT, D_MODEL), lambda i: (i, 0)),
            pl.BlockSpec((N_KV, GH * HEAD_DIM, D_MODEL), lambda i: (0, 0, 0)),
            pl.BlockSpec((1, D_MODEL), lambda i: (0, 0)),
            pl.BlockSpec((D_MODEL, PF_INNER), lambda i: (0, 0)),
            pl.BlockSpec((PF_INNER, D_MODEL), lambda i: (0, 0)),
            pl.BlockSpec((1, D_MODEL), lambda i: (0, 0)),
        ],
        out_specs=[
            pl.BlockSpec((OPF_TT, D_MODEL), lambda i: (i, 0)),
            pl.BlockSpec((OPF_TT, D_MODEL), lambda i: (i, 0)),
            pl.BlockSpec((1, 1, 128), lambda i: (i, 0, 0)),
        ],
        out_shape=[
            jax.ShapeDtypeStruct((N_TOK, D_MODEL), F32),
            jax.ShapeDtypeStruct((N_TOK, D_MODEL), BF16),
            jax.ShapeDtypeStruct((nt, 1, 128), F32),
        ],
        compiler_params=pltpu.CompilerParams(
            dimension_semantics=("parallel",)),
    )(ao, x2d, Wo.astype(BF16).reshape(N_KV, GH * HEAD_DIM, D_MODEL),
      ln_pf_w.reshape(1, D_MODEL), pf_w1.astype(BF16), pf_w2.astype(BF16),
      ln_moe_w.reshape(1, D_MODEL))


def _router_kernel(cv_ref, rwt_ref, rb_ref, w_ref, i_ref):
    logits = _dot(cv_ref[...], rwt_ref[...]) + rb_ref[...]
    m = jnp.max(logits, axis=-1, keepdims=True)
    e = jnp.exp(logits - m)
    w = e / jnp.sum(e, axis=-1, keepdims=True)
    iota = jax.lax.broadcasted_iota(jnp.int32, (B, N_EXPERTS), 1)
    vals = []
    idxs = []
    for _ in range(TOP_K):
        vals.append(jnp.max(w, axis=-1, keepdims=True))
        am = jnp.argmax(w, axis=-1)
        idxs.append(am.astype(jnp.int32))
        w = jnp.where(iota == am[:, None], -1.0, w)
    v = jnp.concatenate(vals, axis=1)
    v = v / jnp.sum(v, axis=1, keepdims=True)
    w_ref[...] = v
    i_ref[...] = jnp.stack(idxs, axis=1)


def _router(cv, router_w, router_b):
    return pl.pallas_call(
        _router_kernel,
        in_specs=[
            pl.BlockSpec((B, N_EXPERTS), lambda: (0, 0)),
            pl.BlockSpec((N_EXPERTS, N_EXPERTS), lambda: (0, 0)),
            pl.BlockSpec((1, N_EXPERTS), lambda: (0, 0)),
        ],
        out_specs=[
            pl.BlockSpec((B, TOP_K), lambda: (0, 0)),
            pl.BlockSpec((B, TOP_K), lambda: (0, 0)),
        ],
        out_shape=[
            jax.ShapeDtypeStruct((B, TOP_K), F32),
            jax.ShapeDtypeStruct((B, TOP_K), jnp.int32),
        ],
    )(cv, router_w.T, router_b.reshape(1, N_EXPERTS))


MOE_IC = 256
NIC = D_INNER // MOE_IC


def _moe_kernel(idx_ref, ws_ref, h_ref, x2_ref, g_ref, u_ref, d_ref, o_ref):
    bb = pl.program_id(0)
    kk = pl.program_id(1)
    ic = pl.program_id(2)
    w = ws_ref[bb * TOP_K + kk]
    h = h_ref[...]
    g = _dot(h, g_ref[0].astype(BF16))
    u = _dot(h, u_ref[0].astype(BF16))
    hid = (_silu(g) * u).astype(BF16)
    contrib = _dot(hid, d_ref[0].astype(BF16)) * w
    first = (kk == 0) & (ic == 0)

    @pl.when(first)
    def _():
        o_ref[...] = x2_ref[...] + contrib

    @pl.when(jnp.logical_not(first))
    def _():
        o_ref[...] = o_ref[...] + contrib


def _moe(idx_slot, w_slot, h2d, x2d, gate_w, up_w, down_w):
    grid_spec = pltpu.PrefetchScalarGridSpec(
        num_scalar_prefetch=2,
        grid=(B, TOP_K, NIC),
        in_specs=[
            pl.BlockSpec((T, D_MODEL), lambda bb, k, ic, idx, ws: (bb, 0)),
            pl.BlockSpec((T, D_MODEL), lambda bb, k, ic, idx, ws: (bb, 0)),
            pl.BlockSpec((1, D_MODEL, MOE_IC),
                         lambda bb, k, ic, idx, ws: (idx[bb * TOP_K + k], 0, ic)),
            pl.BlockSpec((1, D_MODEL, MOE_IC),
                         lambda bb, k, ic, idx, ws: (idx[bb * TOP_K + k], 0, ic)),
            pl.BlockSpec((1, MOE_IC, D_MODEL),
                         lambda bb, k, ic, idx, ws: (idx[bb * TOP_K + k], ic, 0)),
        ],
        out_specs=pl.BlockSpec((T, D_MODEL), lambda bb, k, ic, idx, ws: (bb, 0)),
    )
    return pl.pallas_call(
        _moe_kernel,
        grid_spec=grid_spec,
        out_shape=jax.ShapeDtypeStruct((N_TOK, D_MODEL), F32),
        compiler_params=pltpu.CompilerParams(
            dimension_semantics=("parallel", "arbitrary", "arbitrary")),
    )(idx_slot, w_slot, h2d, x2d, gate_w, up_w, down_w)


@functools.partial(jax.jit, static_argnums=())
def kernel(x, consciousness_vector, ln_attn_w, Wq, Wk, Wv, Wo, ln_pf_w,
           pf_w1, pf_w2, ln_moe_w, gate_w, up_w, down_w, router_w, router_b):
    x2d = x.reshape(N_TOK, D_MODEL)

    q, k, v = _qkv(x2d, ln_attn_w, Wq, Wk, Wv)
    ao = _attention(q, k, v)

    x2, h, t_parts = _opf(ao, x2d, Wo, ln_pf_w, pf_w1, pf_w2, ln_moe_w)
    tension = jnp.sum(t_parts[:, 0, 0]) / (N_TOK * PF_INNER)

    w_sel, idx_sel = _router(consciousness_vector, router_w, router_b)
    idx_slot = idx_sel.reshape(B * TOP_K)
    w_slot = w_sel.reshape(B * TOP_K)

    out = _moe(idx_slot, w_slot, h, x2, gate_w, up_w, down_w)
    return out.reshape(B, T, D_MODEL), tension
